# baseline (speedup 1.0000x reference)
import functools

import jax
import jax.numpy as jnp
from jax import lax
from jax.experimental import pallas as pl
from jax.experimental.pallas import tpu as pltpu
from jax.experimental.pallas import tpu_sc as plsc

_N = 10000
_E = 320000
_P = 8192
_EPS_BN = 1e-5
_EPS_NORM = 1e-12

_NC = 2
_NS = 16
_CH = 128
_EPAD = 327680
_EROWS = _EPAD // _CH
_NACC = 10016
_DUMP = 80
_NCHK = _N // _DUMP
_NPAD = 10240
_DCH = _NPAD // _NS


def _zero_rows(ref, nrows, ncols):
    def body(r, _):
        for k in range(ncols // 16):
            ref[r, pl.ds(k * 16, 16)] = jnp.zeros((16,), jnp.float32)
        return 0
    lax.fori_loop(0, nrows, body, 0)


def _seg_sum_body(split, with_deg, *refs):
    if with_deg:
        (tab_a, tab_b, src, dst, acc_out, deg_out,
         sbuf0, sbuf1, dbuf0, dbuf1, rows0, rows1, onesb, degbuf,
         acc_sh, deg_sh,
         semi0, semi1, semg0, semg1, sems0, sems1, semd0, semd1) = refs
    else:
        (tab_a, tab_b, src, dst, acc_out,
         sbuf0, sbuf1, dbuf0, dbuf1, rows0, rows1,
         acc_sh,
         semi0, semi1, semg0, semg1, sems0, sems1) = refs
    sbuf = [sbuf0, sbuf1]
    dbuf = [dbuf0, dbuf1]
    rows = [rows0, rows1]
    semi = [semi0, semi1]
    semg = [semg0, semg1]
    sems = [sems0, sems1]
    if with_deg:
        semd = [semd0, semd1]
    c = lax.axis_index("c")
    s = lax.axis_index("s")

    def for_each_chunk(fn):
        def body(g, _):
            cid = s + g * _NS
            @pl.when(cid < _NCHK)
            def _():
                fn(cid * _DUMP)
            return 0
        lax.fori_loop(0, (_NCHK + _NS - 1) // _NS, body, 0)

    _zero_rows(rows0, _CH, 128)
    zslice = rows0.at[pl.ds(0, _DUMP)]
    if with_deg:
        for k in range(_DCH // 16):
            degbuf[pl.ds(k * 16, 16)] = jnp.zeros((16,), jnp.float32)
        for k in range(_CH // 16):
            onesb[pl.ds(k * 16, 16)] = jnp.ones((16,), jnp.float32)
        pltpu.sync_copy(degbuf, deg_sh.at[pl.ds(s * _DCH, _DCH)])

    def zero_chunk(off):
        pltpu.sync_copy(zslice, acc_sh.at[pl.ds(off, _DUMP)])
    for_each_chunk(zero_chunk)
    plsc.subcore_barrier()

    edges_per_core = _EPAD // 2 if split else _EPAD
    edges_per_worker = edges_per_core // _NS
    nsuper = edges_per_worker // (8 * _CH)
    base_edge = (c * edges_per_core if split else 0) + s * edges_per_worker

    def run_edges(tab):
        def idx_load(m, b):
            e0 = base_edge + m * 8 * _CH
            pltpu.async_copy(src.at[pl.ds(e0, 8 * _CH)], sbuf[b], semi[b])
            pltpu.async_copy(dst.at[pl.ds(e0, 8 * _CH)], dbuf[b], semi[b])

        def idx_wait(m, b):
            e0 = base_edge + m * 8 * _CH
            pltpu.make_async_copy(src.at[pl.ds(e0, 8 * _CH)], sbuf[b], semi[b]).wait()
            pltpu.make_async_copy(dst.at[pl.ds(e0, 8 * _CH)], dbuf[b], semi[b]).wait()

        idx_load(0, 0)

        def super_body(m2, _):
            for b in range(2):
                m = m2 * 2 + b
                idx_wait(m, b)
                @pl.when(m < nsuper - 1)
                def _():
                    idx_load(m + 1, 1 - b)
                descs = {}
                for j in range(8):
                    rs = j % 2
                    if j >= 2:
                        descs[("s", rs)].wait()
                        if with_deg:
                            descs[("d", rs)].wait()
                    descs[("g", rs)] = pltpu.async_copy(
                        tab.at[sbuf[b].at[pl.ds(j * _CH, _CH)]], rows[rs],
                        semg[rs])
                    if j >= 1:
                        po = 1 - rs
                        descs[("g", po)].wait()
                        descs[("s", po)] = pltpu.async_copy(
                            rows[po],
                            acc_sh.at[dbuf[b].at[pl.ds((j - 1) * _CH, _CH)]],
                            sems[po], add=True)
                        if with_deg:
                            descs[("d", po)] = pltpu.async_copy(
                                onesb,
                                deg_sh.at[dbuf[b].at[pl.ds((j - 1) * _CH, _CH)]],
                                semd[po], add=True)
                descs[("g", 1)].wait()
                descs[("s", 1)] = pltpu.async_copy(
                    rows[1], acc_sh.at[dbuf[b].at[pl.ds(7 * _CH, _CH)]],
                    sems[1], add=True)
                if with_deg:
                    descs[("d", 1)] = pltpu.async_copy(
                        onesb, deg_sh.at[dbuf[b].at[pl.ds(7 * _CH, _CH)]],
                        semd[1], add=True)
                descs[("s", 0)].wait()
                descs[("s", 1)].wait()
                if with_deg:
                    descs[("d", 0)].wait()
                    descs[("d", 1)].wait()
            return 0
        lax.fori_loop(0, nsuper // 2, super_body, 0)

    if split:
        run_edges(tab_a)
    else:
        @pl.when(c == 0)
        def _():
            run_edges(tab_a)
        @pl.when(c == 1)
        def _():
            run_edges(tab_b)

    plsc.subcore_barrier()

    def dump_chunk(off):
        pltpu.sync_copy(acc_sh.at[pl.ds(off, _DUMP)], zslice)
        pltpu.sync_copy(zslice, acc_out.at[c].at[pl.ds(off, _DUMP)])
    for_each_chunk(dump_chunk)
    if with_deg:
        pltpu.sync_copy(deg_sh.at[pl.ds(s * _DCH, _DCH)], degbuf)
        pltpu.sync_copy(degbuf, deg_out.at[c].at[pl.ds(s * _DCH, _DCH)])


def _make_seg_sum(split, with_deg):
    mesh = plsc.VectorSubcoreMesh(
        core_axis_name="c", subcore_axis_name="s",
        num_cores=_NC, num_subcores=_NS)
    out_type = [jax.ShapeDtypeStruct((_NC, _N, 128), jnp.float32)]
    scratch = [
        pltpu.VMEM((8 * _CH,), jnp.int32),
        pltpu.VMEM((8 * _CH,), jnp.int32),
        pltpu.VMEM((8 * _CH,), jnp.int32),
        pltpu.VMEM((8 * _CH,), jnp.int32),
        pltpu.VMEM((_CH, 128), jnp.float32),
        pltpu.VMEM((_CH, 128), jnp.float32),
    ]
    if with_deg:
        out_type.append(jax.ShapeDtypeStruct((_NC, _NPAD), jnp.float32))
        scratch.append(pltpu.VMEM((_CH,), jnp.float32))
        scratch.append(pltpu.VMEM((_DCH,), jnp.float32))
    scratch.append(pltpu.VMEM_SHARED((_NACC, 128), jnp.float32))
    if with_deg:
        scratch.append(pltpu.VMEM_SHARED((_NPAD,), jnp.float32))
    nsem = 8 if with_deg else 6
    scratch.extend([pltpu.SemaphoreType.DMA] * nsem)
    return pl.kernel(
        functools.partial(_seg_sum_body, split, with_deg),
        out_type=tuple(out_type) if len(out_type) > 1 else out_type[0],
        mesh=mesh,
        scratch_types=tuple(scratch),
        name=f"sc_seg_sum_split{int(split)}_deg{int(with_deg)}",
    )


def _pair_gather_body(emb, idx, out, sidx, rows, gsem):
    c = lax.axis_index("c")
    s = lax.axis_index("s")
    wid = s * _NC + c
    rows_per_worker = (2 * _P) // (_NC * _NS)
    ch = 64
    def body(g, _):
        off = wid * rows_per_worker + g * ch
        pltpu.sync_copy(idx.at[pl.ds(off, ch)], sidx)
        pltpu.async_copy(emb.at[sidx], rows, gsem).wait()
        pltpu.sync_copy(rows, out.at[pl.ds(off, ch)])
        return 0
    lax.fori_loop(0, rows_per_worker // ch, body, 0)


def _make_pair_gather():
    mesh = plsc.VectorSubcoreMesh(
        core_axis_name="c", subcore_axis_name="s",
        num_cores=_NC, num_subcores=_NS)
    return pl.kernel(
        _pair_gather_body,
        out_type=jax.ShapeDtypeStruct((2 * _P, 128), jnp.float32),
        mesh=mesh,
        scratch_types=(
            pltpu.VMEM((64,), jnp.int32),
            pltpu.VMEM((64, 128), jnp.float32),
            pltpu.SemaphoreType.DMA,
        ),
        name="sc_pair_gather",
    )


_BR = 1000
_G = _N // _BR


def _tc_layer_in(acc_ref, degp_ref, x_ref, wl_ref, wr_ref, b_ref,
                 hpre_ref, dinv_ref, stats_ref, sacc_ref):
    i = pl.program_id(0)
    deg = degp_ref[0] + degp_ref[1]
    dinv = 1.0 / jnp.maximum(deg, 1.0)
    agg = (acc_ref[0] + acc_ref[1]) * dinv
    hpre = (jnp.dot(agg, wl_ref[...], preferred_element_type=jnp.float32)
            + jnp.dot(x_ref[...], wr_ref[...], preferred_element_type=jnp.float32)
            + b_ref[...])
    hpre_ref[...] = hpre
    dinv_ref[...] = dinv
    @pl.when(i == 0)
    def _():
        sacc_ref[...] = jnp.zeros_like(sacc_ref)
    sacc_ref[0, :] += jnp.sum(hpre, axis=0)
    sacc_ref[1, :] += jnp.sum(hpre * hpre, axis=0)
    @pl.when(i == _G - 1)
    def _():
        stats_ref[...] = sacc_ref[...]


def _tc_layer_mid(acc_ref, dinv_ref, h_ref, wl_ref, wr_ref, b_ref,
                  hpre_ref, stats_ref, sacc_ref):
    i = pl.program_id(0)
    dinv = dinv_ref[...]
    a0 = acc_ref[0] * dinv
    a1 = acc_ref[1] * dinv
    hpre = (jnp.dot(a0, wl_ref[0:128, :], preferred_element_type=jnp.float32)
            + jnp.dot(a1, wl_ref[128:256, :], preferred_element_type=jnp.float32)
            + jnp.dot(h_ref[0], wr_ref[0:128, :], preferred_element_type=jnp.float32)
            + jnp.dot(h_ref[1], wr_ref[128:256, :], preferred_element_type=jnp.float32)
            + b_ref[...])
    hpre_ref[...] = hpre
    @pl.when(i == 0)
    def _():
        sacc_ref[...] = jnp.zeros_like(sacc_ref)
    sacc_ref[0, :] += jnp.sum(hpre, axis=0)
    sacc_ref[1, :] += jnp.sum(hpre * hpre, axis=0)
    @pl.when(i == _G - 1)
    def _():
        stats_ref[...] = sacc_ref[...]


def _tc_layer_out(acc_ref, dinv_ref, h_ref, wr_ref, b_ref,
                  hpre_ref, stats_ref, sacc_ref):
    i = pl.program_id(0)
    agg = (acc_ref[0] + acc_ref[1]) * dinv_ref[...]
    hpre = (agg
            + jnp.dot(h_ref[0], wr_ref[0:128, :], preferred_element_type=jnp.float32)
            + jnp.dot(h_ref[1], wr_ref[128:256, :], preferred_element_type=jnp.float32)
            + b_ref[...])
    hpre_ref[...] = hpre
    @pl.when(i == 0)
    def _():
        sacc_ref[...] = jnp.zeros_like(sacc_ref)
    sacc_ref[0, :] += jnp.sum(hpre, axis=0)
    sacc_ref[1, :] += jnp.sum(hpre * hpre, axis=0)
    @pl.when(i == _G - 1)
    def _():
        stats_ref[...] = sacc_ref[...]


def _tc_bn_relu_split(hpre_ref, stats_ref, g_ref, be_ref, out_ref):
    mean = stats_ref[0:1, :] / _N
    var = stats_ref[1:2, :] / _N - mean * mean
    scale = g_ref[...] * lax.rsqrt(var + _EPS_BN)
    shift = be_ref[...] - mean * scale
    h = jnp.maximum(hpre_ref[...] * scale + shift, 0.0)
    out_ref[0] = h[:, 0:128]
    out_ref[1] = h[:, 128:256]


def _tc_bn_relu_split_proj(hpre_ref, stats_ref, g_ref, be_ref, wl_ref,
                           out_ref, p_ref):
    mean = stats_ref[0:1, :] / _N
    var = stats_ref[1:2, :] / _N - mean * mean
    scale = g_ref[...] * lax.rsqrt(var + _EPS_BN)
    shift = be_ref[...] - mean * scale
    h = jnp.maximum(hpre_ref[...] * scale + shift, 0.0)
    out_ref[0] = h[:, 0:128]
    out_ref[1] = h[:, 128:256]
    p_ref[...] = jnp.dot(h, wl_ref[...], preferred_element_type=jnp.float32)


def _tc_bn_l2norm(hpre_ref, stats_ref, g_ref, be_ref, emb_ref):
    mean = stats_ref[0:1, :] / _N
    var = stats_ref[1:2, :] / _N - mean * mean
    scale = g_ref[...] * lax.rsqrt(var + _EPS_BN)
    shift = be_ref[...] - mean * scale
    z = hpre_ref[...] * scale + shift
    rn = jnp.sqrt(jnp.sum(z * z, axis=-1, keepdims=True))
    emb_ref[...] = z / jnp.maximum(rn, _EPS_NORM)


def _tc_scores(a_ref, b_ref, s_ref):
    s_ref[...] = jnp.sum(a_ref[...] * b_ref[...], axis=-1, keepdims=True)


def _row_spec(shape3=None, d=256):
    return pl.BlockSpec((_BR, d), lambda i: (i, 0))


def kernel(x, edge_index, src_indices, dst_indices,
           Wl0, Wr0, b0, g0, be0,
           Wl1, Wr1, b1, g1, be1,
           Wl2, Wr2, b2, g2, be2):
    src = edge_index[0].astype(jnp.int32)
    dst = edge_index[1].astype(jnp.int32)
    npad = _EPAD - _E
    src1d = jnp.concatenate([src, jnp.zeros((npad,), jnp.int32)])
    dst1d = jnp.concatenate([dst, jnp.full((npad,), _N, jnp.int32)])
    pair_idx = jnp.concatenate([src_indices.astype(jnp.int32),
                                dst_indices.astype(jnp.int32)])
    b0r, g0r, be0r = b0.reshape(1, -1), g0.reshape(1, -1), be0.reshape(1, -1)
    b1r, g1r, be1r = b1.reshape(1, -1), g1.reshape(1, -1), be1.reshape(1, -1)
    b2r, g2r, be2r = b2.reshape(1, -1), g2.reshape(1, -1), be2.reshape(1, -1)

    seg_deg = _make_seg_sum(split=True, with_deg=True)
    seg_split = _make_seg_sum(split=True, with_deg=False)
    seg_stack = _make_seg_sum(split=False, with_deg=False)
    pair_gather = _make_pair_gather()

    full = lambda shp: pl.BlockSpec(shp, lambda i: tuple(0 for _ in shp))
    acc_spec = pl.BlockSpec((2, _BR, 128), lambda i: (0, i, 0))
    h2_spec = pl.BlockSpec((2, _BR, 128), lambda i: (0, i, 0))
    cp_arb = pltpu.CompilerParams(dimension_semantics=("arbitrary",))

    acc0, degp = seg_deg(x, x, src1d, dst1d)
    degp = degp[:, :_N].reshape(2, _N, 1)

    hpre1, dinv, stats1 = pl.pallas_call(
        _tc_layer_in,
        grid=(_G,),
        in_specs=[acc_spec,
                  pl.BlockSpec((2, _BR, 1), lambda i: (0, i, 0)),
                  pl.BlockSpec((_BR, 128), lambda i: (i, 0)),
                  full((128, 256)), full((128, 256)), full((1, 256))],
        out_specs=[pl.BlockSpec((_BR, 256), lambda i: (i, 0)),
                   pl.BlockSpec((_BR, 1), lambda i: (i, 0)),
                   full((2, 256))],
        out_shape=[jax.ShapeDtypeStruct((_N, 256), jnp.float32),
                   jax.ShapeDtypeStruct((_N, 1), jnp.float32),
                   jax.ShapeDtypeStruct((2, 256), jnp.float32)],
        scratch_shapes=[pltpu.VMEM((2, 256), jnp.float32)],
        compiler_params=cp_arb,
        name="tc_layer0",
    )(acc0, degp, x, Wl0, Wr0, b0r)

    h1s = pl.pallas_call(
        _tc_bn_relu_split,
        grid=(_G,),
        in_specs=[pl.BlockSpec((_BR, 256), lambda i: (i, 0)),
                  full((2, 256)), full((1, 256)), full((1, 256))],
        out_specs=h2_spec,
        out_shape=jax.ShapeDtypeStruct((2, _N, 128), jnp.float32),
        name="tc_bn0",
    )(hpre1, stats1, g0r, be0r)

    acc1 = seg_stack(h1s[0], h1s[1], src1d, dst1d)

    hpre2, stats2 = pl.pallas_call(
        _tc_layer_mid,
        grid=(_G,),
        in_specs=[acc_spec,
                  pl.BlockSpec((_BR, 1), lambda i: (i, 0)),
                  h2_spec,
                  full((256, 256)), full((256, 256)), full((1, 256))],
        out_specs=[pl.BlockSpec((_BR, 256), lambda i: (i, 0)),
                   full((2, 256))],
        out_shape=[jax.ShapeDtypeStruct((_N, 256), jnp.float32),
                   jax.ShapeDtypeStruct((2, 256), jnp.float32)],
        scratch_shapes=[pltpu.VMEM((2, 256), jnp.float32)],
        compiler_params=cp_arb,
        name="tc_layer1",
    )(acc1, dinv, h1s, Wl1, Wr1, b1r)

    h2s, p = pl.pallas_call(
        _tc_bn_relu_split_proj,
        grid=(_G,),
        in_specs=[pl.BlockSpec((_BR, 256), lambda i: (i, 0)),
                  full((2, 256)), full((1, 256)), full((1, 256)),
                  full((256, 128))],
        out_specs=[h2_spec, pl.BlockSpec((_BR, 128), lambda i: (i, 0))],
        out_shape=[jax.ShapeDtypeStruct((2, _N, 128), jnp.float32),
                   jax.ShapeDtypeStruct((_N, 128), jnp.float32)],
        name="tc_bn1",
    )(hpre2, stats2, g1r, be1r, Wl2)

    acc2 = seg_split(p, p, src1d, dst1d)

    hpre3, stats3 = pl.pallas_call(
        _tc_layer_out,
        grid=(_G,),
        in_specs=[acc_spec,
                  pl.BlockSpec((_BR, 1), lambda i: (i, 0)),
                  h2_spec,
                  full((256, 128)), full((1, 128))],
        out_specs=[pl.BlockSpec((_BR, 128), lambda i: (i, 0)),
                   full((2, 128))],
        out_shape=[jax.ShapeDtypeStruct((_N, 128), jnp.float32),
                   jax.ShapeDtypeStruct((2, 128), jnp.float32)],
        scratch_shapes=[pltpu.VMEM((2, 128), jnp.float32)],
        compiler_params=cp_arb,
        name="tc_layer2",
    )(acc2, dinv, h2s, Wr2, b2r)

    emb = pl.pallas_call(
        _tc_bn_l2norm,
        grid=(_G,),
        in_specs=[pl.BlockSpec((_BR, 128), lambda i: (i, 0)),
                  full((2, 128)), full((1, 128)), full((1, 128))],
        out_specs=pl.BlockSpec((_BR, 128), lambda i: (i, 0)),
        out_shape=jax.ShapeDtypeStruct((_N, 128), jnp.float32),
        name="tc_bn2_l2",
    )(hpre3, stats3, g2r, be2r)

    pairs = pair_gather(emb, pair_idx)

    pb = 1024
    scores2d = pl.pallas_call(
        _tc_scores,
        grid=(_P // pb,),
        in_specs=[pl.BlockSpec((pb, 128), lambda i: (i, 0)),
                  pl.BlockSpec((pb, 128), lambda i: (i + _P // pb, 0))],
        out_specs=pl.BlockSpec((pb, 1), lambda i: (i, 0)),
        out_shape=jax.ShapeDtypeStruct((_P, 1), jnp.float32),
        name="tc_scores",
    )(pairs, pairs)
    scores = scores2d[:, 0]

    return (emb, scores)


# speedup vs baseline: 1.0220x; 1.0220x over previous
import functools

import jax
import jax.numpy as jnp
from jax import lax
from jax.experimental import pallas as pl
from jax.experimental.pallas import tpu as pltpu
from jax.experimental.pallas import tpu_sc as plsc

_N = 10000
_E = 320000
_P = 8192
_EPS_BN = 1e-5
_EPS_NORM = 1e-12

_NC = 2
_NS = 16
_CH = 128
_EPAD = 327680
_EROWS = _EPAD // _CH
_NACC = 10240
_DUMP = 80
_NCHK = _N // _DUMP
_NPAD = 10240
_DCH = _NPAD // _NS


def _zero_rows(ref, nrows, ncols):
    def body(r, _):
        for k in range(ncols // 16):
            ref[r, pl.ds(k * 16, 16)] = jnp.zeros((16,), jnp.float32)
        return 0
    lax.fori_loop(0, nrows, body, 0)


def _seg_sum_body(split, with_deg, *refs):
    if with_deg:
        (tab_a, tab_b, src, dst, acc_out, deg_out,
         sbuf0, sbuf1, dbuf0, dbuf1, rows0, rows1, onesb, degbuf,
         acc_sh, deg_sh,
         semi0, semi1, semg0, semg1, sems0, sems1, semd0, semd1) = refs
    else:
        (tab_a, tab_b, src, dst, acc_out,
         sbuf0, sbuf1, dbuf0, dbuf1, rows0, rows1,
         acc_sh,
         semi0, semi1, semg0, semg1, sems0, sems1) = refs
    sbuf = [sbuf0, sbuf1]
    dbuf = [dbuf0, dbuf1]
    rows = [rows0, rows1]
    semi = [semi0, semi1]
    semg = [semg0, semg1]
    sems = [sems0, sems1]
    if with_deg:
        semd = [semd0, semd1]
    c = lax.axis_index("c")
    s = lax.axis_index("s")

    def for_each_chunk(fn):
        def body(g, _):
            cid = s + g * _NS
            @pl.when(cid < _NCHK)
            def _():
                fn(cid * _DUMP)
            return 0
        lax.fori_loop(0, (_NCHK + _NS - 1) // _NS, body, 0)

    _zero_rows(rows0, _CH, 128)
    zslice = rows0.at[pl.ds(0, _DUMP)]
    if with_deg:
        for k in range(_DCH // 16):
            degbuf[pl.ds(k * 16, 16)] = jnp.zeros((16,), jnp.float32)
        for k in range(_CH // 16):
            onesb[pl.ds(k * 16, 16)] = jnp.ones((16,), jnp.float32)
        pltpu.sync_copy(degbuf, deg_sh.at[pl.ds(s * _DCH, _DCH)])

    def zero_chunk(off):
        pltpu.sync_copy(zslice, acc_sh.at[pl.ds(off, _DUMP)])
    for_each_chunk(zero_chunk)
    plsc.subcore_barrier()

    edges_per_core = _EPAD // 2 if split else _EPAD
    edges_per_worker = edges_per_core // _NS
    nsuper = edges_per_worker // (8 * _CH)
    base_edge = (c * edges_per_core if split else 0) + s * edges_per_worker

    def run_edges(tab):
        def idx_load(m, b):
            e0 = base_edge + m * 8 * _CH
            pltpu.async_copy(src.at[pl.ds(e0, 8 * _CH)], sbuf[b], semi[b])
            pltpu.async_copy(dst.at[pl.ds(e0, 8 * _CH)], dbuf[b], semi[b])

        def idx_wait(m, b):
            e0 = base_edge + m * 8 * _CH
            pltpu.make_async_copy(src.at[pl.ds(e0, 8 * _CH)], sbuf[b], semi[b]).wait()
            pltpu.make_async_copy(dst.at[pl.ds(e0, 8 * _CH)], dbuf[b], semi[b]).wait()

        idx_load(0, 0)

        def super_body(m2, _):
            for b in range(2):
                m = m2 * 2 + b
                idx_wait(m, b)
                @pl.when(m < nsuper - 1)
                def _():
                    idx_load(m + 1, 1 - b)
                descs = {}
                for j in range(8):
                    rs = j % 2
                    if j >= 2:
                        descs[("s", rs)].wait()
                        if with_deg:
                            descs[("d", rs)].wait()
                    descs[("g", rs)] = pltpu.async_copy(
                        tab.at[sbuf[b].at[pl.ds(j * _CH, _CH)]], rows[rs],
                        semg[rs])
                    if j >= 1:
                        po = 1 - rs
                        descs[("g", po)].wait()
                        descs[("s", po)] = pltpu.async_copy(
                            rows[po],
                            acc_sh.at[dbuf[b].at[pl.ds((j - 1) * _CH, _CH)]],
                            sems[po], add=True)
                        if with_deg:
                            descs[("d", po)] = pltpu.async_copy(
                                onesb,
                                deg_sh.at[dbuf[b].at[pl.ds((j - 1) * _CH, _CH)]],
                                semd[po], add=True)
                descs[("g", 1)].wait()
                descs[("s", 1)] = pltpu.async_copy(
                    rows[1], acc_sh.at[dbuf[b].at[pl.ds(7 * _CH, _CH)]],
                    sems[1], add=True)
                if with_deg:
                    descs[("d", 1)] = pltpu.async_copy(
                        onesb, deg_sh.at[dbuf[b].at[pl.ds(7 * _CH, _CH)]],
                        semd[1], add=True)
                descs[("s", 0)].wait()
                descs[("s", 1)].wait()
                if with_deg:
                    descs[("d", 0)].wait()
                    descs[("d", 1)].wait()
            return 0
        lax.fori_loop(0, nsuper // 2, super_body, 0)

    if split:
        run_edges(tab_a)
    else:
        @pl.when(c == 0)
        def _():
            run_edges(tab_a)
        @pl.when(c == 1)
        def _():
            run_edges(tab_b)

    plsc.subcore_barrier()

    def dump_chunk(off):
        pltpu.sync_copy(acc_sh.at[pl.ds(off, _DUMP)], zslice)
        pltpu.sync_copy(zslice, acc_out.at[c].at[pl.ds(off, _DUMP)])
    for_each_chunk(dump_chunk)
    if with_deg:
        pltpu.sync_copy(deg_sh.at[pl.ds(s * _DCH, _DCH)], degbuf)
        pltpu.sync_copy(degbuf, deg_out.at[c].at[pl.ds(s * _DCH, _DCH)])


def _make_seg_sum(split, with_deg):
    mesh = plsc.VectorSubcoreMesh(
        core_axis_name="c", subcore_axis_name="s",
        num_cores=_NC, num_subcores=_NS)
    out_type = [jax.ShapeDtypeStruct((_NC, _N, 128), jnp.float32)]
    scratch = [
        pltpu.VMEM((8 * _CH,), jnp.int32),
        pltpu.VMEM((8 * _CH,), jnp.int32),
        pltpu.VMEM((8 * _CH,), jnp.int32),
        pltpu.VMEM((8 * _CH,), jnp.int32),
        pltpu.VMEM((_CH, 128), jnp.float32),
        pltpu.VMEM((_CH, 128), jnp.float32),
    ]
    if with_deg:
        out_type.append(jax.ShapeDtypeStruct((_NC, _NPAD), jnp.float32))
        scratch.append(pltpu.VMEM((_CH,), jnp.float32))
        scratch.append(pltpu.VMEM((_DCH,), jnp.float32))
    scratch.append(pltpu.VMEM_SHARED((_NACC, 128), jnp.float32))
    if with_deg:
        scratch.append(pltpu.VMEM_SHARED((_NPAD,), jnp.float32))
    nsem = 8 if with_deg else 6
    scratch.extend([pltpu.SemaphoreType.DMA] * nsem)
    return pl.kernel(
        functools.partial(_seg_sum_body, split, with_deg),
        out_type=tuple(out_type) if len(out_type) > 1 else out_type[0],
        mesh=mesh,
        scratch_types=tuple(scratch),
        name=f"sc_seg_sum_split{int(split)}_deg{int(with_deg)}",
    )


def _pair_gather_body(emb, idx, out, sidx, rows, gsem):
    c = lax.axis_index("c")
    s = lax.axis_index("s")
    wid = s * _NC + c
    rows_per_worker = (2 * _P) // (_NC * _NS)
    ch = 64
    def body(g, _):
        off = wid * rows_per_worker + g * ch
        pltpu.sync_copy(idx.at[pl.ds(off, ch)], sidx)
        pltpu.async_copy(emb.at[sidx], rows, gsem).wait()
        pltpu.sync_copy(rows, out.at[pl.ds(off, ch)])
        return 0
    lax.fori_loop(0, rows_per_worker // ch, body, 0)


def _make_pair_gather():
    mesh = plsc.VectorSubcoreMesh(
        core_axis_name="c", subcore_axis_name="s",
        num_cores=_NC, num_subcores=_NS)
    return pl.kernel(
        _pair_gather_body,
        out_type=jax.ShapeDtypeStruct((2 * _P, 128), jnp.float32),
        mesh=mesh,
        scratch_types=(
            pltpu.VMEM((64,), jnp.int32),
            pltpu.VMEM((64, 128), jnp.float32),
            pltpu.SemaphoreType.DMA,
        ),
        name="sc_pair_gather",
    )


_BR = 1000
_G = _N // _BR


def _tc_layer_in(acc_ref, degp_ref, x_ref, wl_ref, wr_ref, b_ref,
                 hpre_ref, dinv_ref, stats_ref, sacc_ref):
    i = pl.program_id(0)
    deg = degp_ref[0] + degp_ref[1]
    dinv = 1.0 / jnp.maximum(deg, 1.0)
    agg = (acc_ref[0] + acc_ref[1]) * dinv
    hpre = (jnp.dot(agg, wl_ref[...], preferred_element_type=jnp.float32)
            + jnp.dot(x_ref[...], wr_ref[...], preferred_element_type=jnp.float32)
            + b_ref[...])
    hpre_ref[...] = hpre
    dinv_ref[...] = dinv
    @pl.when(i == 0)
    def _():
        sacc_ref[...] = jnp.zeros_like(sacc_ref)
    sacc_ref[0, :] += jnp.sum(hpre, axis=0)
    sacc_ref[1, :] += jnp.sum(hpre * hpre, axis=0)
    @pl.when(i == _G - 1)
    def _():
        stats_ref[...] = sacc_ref[...]


def _tc_layer_mid(acc_ref, dinv_ref, h_ref, wl_ref, wr_ref, b_ref,
                  hpre_ref, stats_ref, sacc_ref):
    i = pl.program_id(0)
    dinv = dinv_ref[...]
    a0 = acc_ref[0] * dinv
    a1 = acc_ref[1] * dinv
    hpre = (jnp.dot(a0, wl_ref[0:128, :], preferred_element_type=jnp.float32)
            + jnp.dot(a1, wl_ref[128:256, :], preferred_element_type=jnp.float32)
            + jnp.dot(h_ref[0], wr_ref[0:128, :], preferred_element_type=jnp.float32)
            + jnp.dot(h_ref[1], wr_ref[128:256, :], preferred_element_type=jnp.float32)
            + b_ref[...])
    hpre_ref[...] = hpre
    @pl.when(i == 0)
    def _():
        sacc_ref[...] = jnp.zeros_like(sacc_ref)
    sacc_ref[0, :] += jnp.sum(hpre, axis=0)
    sacc_ref[1, :] += jnp.sum(hpre * hpre, axis=0)
    @pl.when(i == _G - 1)
    def _():
        stats_ref[...] = sacc_ref[...]


def _tc_layer_out(acc_ref, dinv_ref, h_ref, wr_ref, b_ref,
                  hpre_ref, stats_ref, sacc_ref):
    i = pl.program_id(0)
    agg = (acc_ref[0] + acc_ref[1]) * dinv_ref[...]
    hpre = (agg
            + jnp.dot(h_ref[0], wr_ref[0:128, :], preferred_element_type=jnp.float32)
            + jnp.dot(h_ref[1], wr_ref[128:256, :], preferred_element_type=jnp.float32)
            + b_ref[...])
    hpre_ref[...] = hpre
    @pl.when(i == 0)
    def _():
        sacc_ref[...] = jnp.zeros_like(sacc_ref)
    sacc_ref[0, :] += jnp.sum(hpre, axis=0)
    sacc_ref[1, :] += jnp.sum(hpre * hpre, axis=0)
    @pl.when(i == _G - 1)
    def _():
        stats_ref[...] = sacc_ref[...]


def _tc_bn_relu_split(hpre_ref, stats_ref, g_ref, be_ref, out_ref):
    mean = stats_ref[0:1, :] / _N
    var = stats_ref[1:2, :] / _N - mean * mean
    scale = g_ref[...] * lax.rsqrt(var + _EPS_BN)
    shift = be_ref[...] - mean * scale
    h = jnp.maximum(hpre_ref[...] * scale + shift, 0.0)
    out_ref[0] = h[:, 0:128]
    out_ref[1] = h[:, 128:256]


def _tc_bn_relu_split_proj(hpre_ref, stats_ref, g_ref, be_ref, wl_ref,
                           out_ref, p_ref):
    mean = stats_ref[0:1, :] / _N
    var = stats_ref[1:2, :] / _N - mean * mean
    scale = g_ref[...] * lax.rsqrt(var + _EPS_BN)
    shift = be_ref[...] - mean * scale
    h = jnp.maximum(hpre_ref[...] * scale + shift, 0.0)
    out_ref[0] = h[:, 0:128]
    out_ref[1] = h[:, 128:256]
    p_ref[...] = jnp.dot(h, wl_ref[...], preferred_element_type=jnp.float32)


def _tc_bn_l2norm(hpre_ref, stats_ref, g_ref, be_ref, emb_ref):
    mean = stats_ref[0:1, :] / _N
    var = stats_ref[1:2, :] / _N - mean * mean
    scale = g_ref[...] * lax.rsqrt(var + _EPS_BN)
    shift = be_ref[...] - mean * scale
    z = hpre_ref[...] * scale + shift
    rn = jnp.sqrt(jnp.sum(z * z, axis=-1, keepdims=True))
    emb_ref[...] = z / jnp.maximum(rn, _EPS_NORM)


def _tc_scores(a_ref, b_ref, s_ref):
    s_ref[...] = jnp.sum(a_ref[...] * b_ref[...], axis=-1, keepdims=True)


def _row_spec(shape3=None, d=256):
    return pl.BlockSpec((_BR, d), lambda i: (i, 0))


def kernel(x, edge_index, src_indices, dst_indices,
           Wl0, Wr0, b0, g0, be0,
           Wl1, Wr1, b1, g1, be1,
           Wl2, Wr2, b2, g2, be2):
    src = edge_index[0].astype(jnp.int32)
    dst = edge_index[1].astype(jnp.int32)
    npad = _EPAD - _E
    src1d = jnp.concatenate([src, jnp.zeros((npad,), jnp.int32)])
    pad_dst = _N + (jnp.arange(npad, dtype=jnp.int32) % (_NACC - _N))
    dst1d = jnp.concatenate([dst, pad_dst])
    pair_idx = jnp.concatenate([src_indices.astype(jnp.int32),
                                dst_indices.astype(jnp.int32)])
    b0r, g0r, be0r = b0.reshape(1, -1), g0.reshape(1, -1), be0.reshape(1, -1)
    b1r, g1r, be1r = b1.reshape(1, -1), g1.reshape(1, -1), be1.reshape(1, -1)
    b2r, g2r, be2r = b2.reshape(1, -1), g2.reshape(1, -1), be2.reshape(1, -1)

    seg_deg = _make_seg_sum(split=True, with_deg=True)
    seg_split = _make_seg_sum(split=True, with_deg=False)
    seg_stack = _make_seg_sum(split=False, with_deg=False)
    pair_gather = _make_pair_gather()

    full = lambda shp: pl.BlockSpec(shp, lambda i: tuple(0 for _ in shp))
    acc_spec = pl.BlockSpec((2, _BR, 128), lambda i: (0, i, 0))
    h2_spec = pl.BlockSpec((2, _BR, 128), lambda i: (0, i, 0))
    cp_arb = pltpu.CompilerParams(dimension_semantics=("arbitrary",))

    acc0, degp = seg_deg(x, x, src1d, dst1d)
    degp = degp[:, :_N].reshape(2, _N, 1)

    hpre1, dinv, stats1 = pl.pallas_call(
        _tc_layer_in,
        grid=(_G,),
        in_specs=[acc_spec,
                  pl.BlockSpec((2, _BR, 1), lambda i: (0, i, 0)),
                  pl.BlockSpec((_BR, 128), lambda i: (i, 0)),
                  full((128, 256)), full((128, 256)), full((1, 256))],
        out_specs=[pl.BlockSpec((_BR, 256), lambda i: (i, 0)),
                   pl.BlockSpec((_BR, 1), lambda i: (i, 0)),
                   full((2, 256))],
        out_shape=[jax.ShapeDtypeStruct((_N, 256), jnp.float32),
                   jax.ShapeDtypeStruct((_N, 1), jnp.float32),
                   jax.ShapeDtypeStruct((2, 256), jnp.float32)],
        scratch_shapes=[pltpu.VMEM((2, 256), jnp.float32)],
        compiler_params=cp_arb,
        name="tc_layer0",
    )(acc0, degp, x, Wl0, Wr0, b0r)

    h1s = pl.pallas_call(
        _tc_bn_relu_split,
        grid=(_G,),
        in_specs=[pl.BlockSpec((_BR, 256), lambda i: (i, 0)),
                  full((2, 256)), full((1, 256)), full((1, 256))],
        out_specs=h2_spec,
        out_shape=jax.ShapeDtypeStruct((2, _N, 128), jnp.float32),
        name="tc_bn0",
    )(hpre1, stats1, g0r, be0r)

    acc1 = seg_stack(h1s[0], h1s[1], src1d, dst1d)

    hpre2, stats2 = pl.pallas_call(
        _tc_layer_mid,
        grid=(_G,),
        in_specs=[acc_spec,
                  pl.BlockSpec((_BR, 1), lambda i: (i, 0)),
                  h2_spec,
                  full((256, 256)), full((256, 256)), full((1, 256))],
        out_specs=[pl.BlockSpec((_BR, 256), lambda i: (i, 0)),
                   full((2, 256))],
        out_shape=[jax.ShapeDtypeStruct((_N, 256), jnp.float32),
                   jax.ShapeDtypeStruct((2, 256), jnp.float32)],
        scratch_shapes=[pltpu.VMEM((2, 256), jnp.float32)],
        compiler_params=cp_arb,
        name="tc_layer1",
    )(acc1, dinv, h1s, Wl1, Wr1, b1r)

    h2s, p = pl.pallas_call(
        _tc_bn_relu_split_proj,
        grid=(_G,),
        in_specs=[pl.BlockSpec((_BR, 256), lambda i: (i, 0)),
                  full((2, 256)), full((1, 256)), full((1, 256)),
                  full((256, 128))],
        out_specs=[h2_spec, pl.BlockSpec((_BR, 128), lambda i: (i, 0))],
        out_shape=[jax.ShapeDtypeStruct((2, _N, 128), jnp.float32),
                   jax.ShapeDtypeStruct((_N, 128), jnp.float32)],
        name="tc_bn1",
    )(hpre2, stats2, g1r, be1r, Wl2)

    acc2 = seg_split(p, p, src1d, dst1d)

    hpre3, stats3 = pl.pallas_call(
        _tc_layer_out,
        grid=(_G,),
        in_specs=[acc_spec,
                  pl.BlockSpec((_BR, 1), lambda i: (i, 0)),
                  h2_spec,
                  full((256, 128)), full((1, 128))],
        out_specs=[pl.BlockSpec((_BR, 128), lambda i: (i, 0)),
                   full((2, 128))],
        out_shape=[jax.ShapeDtypeStruct((_N, 128), jnp.float32),
                   jax.ShapeDtypeStruct((2, 128), jnp.float32)],
        scratch_shapes=[pltpu.VMEM((2, 128), jnp.float32)],
        compiler_params=cp_arb,
        name="tc_layer2",
    )(acc2, dinv, h2s, Wr2, b2r)

    emb = pl.pallas_call(
        _tc_bn_l2norm,
        grid=(_G,),
        in_specs=[pl.BlockSpec((_BR, 128), lambda i: (i, 0)),
                  full((2, 128)), full((1, 128)), full((1, 128))],
        out_specs=pl.BlockSpec((_BR, 128), lambda i: (i, 0)),
        out_shape=jax.ShapeDtypeStruct((_N, 128), jnp.float32),
        name="tc_bn2_l2",
    )(hpre3, stats3, g2r, be2r)

    pairs = pair_gather(emb, pair_idx)

    pb = 1024
    scores2d = pl.pallas_call(
        _tc_scores,
        grid=(_P // pb,),
        in_specs=[pl.BlockSpec((pb, 128), lambda i: (i, 0)),
                  pl.BlockSpec((pb, 128), lambda i: (i + _P // pb, 0))],
        out_specs=pl.BlockSpec((pb, 1), lambda i: (i, 0)),
        out_shape=jax.ShapeDtypeStruct((_P, 1), jnp.float32),
        name="tc_scores",
    )(pairs, pairs)
    scores = scores2d[:, 0]

    return (emb, scores)


# speedup vs baseline: 2.9889x; 2.9246x over previous
import functools

import jax
import jax.numpy as jnp
from jax import lax
from jax.experimental import pallas as pl
from jax.experimental.pallas import tpu as pltpu
from jax.experimental.pallas import tpu_sc as plsc

_N = 10000
_E = 320000
_P = 8192
_EPS_BN = 1e-5
_EPS_NORM = 1e-12

_NC = 2
_NS = 16
_CH = 128
_EPAD = 327680
_EROWS = _EPAD // _CH
_NACC = 10240
_DUMP = 80
_NCHK = _N // _DUMP
_NPAD = 10240
_DCH = _NPAD // _NS


def _zero_rows(ref, nrows, ncols):
    def body(r, _):
        for k in range(ncols // 16):
            ref[r, pl.ds(k * 16, 16)] = jnp.zeros((16,), jnp.float32)
        return 0
    lax.fori_loop(0, nrows, body, 0)


def _seg_sum_body(split, with_deg, *refs, _force_half=None):
    if with_deg:
        (tab_a, tab_b, src, dst, acc_out, deg_out,
         sbuf0, sbuf1, dbuf0, dbuf1, rows0, rows1, onesb, degbuf,
         acc_sh, deg_sh,
         semi0, semi1, semg0, semg1, sems0, sems1, semd0, semd1) = refs
    else:
        (tab_a, tab_b, src, dst, acc_out,
         sbuf0, sbuf1, dbuf0, dbuf1, rows0, rows1,
         acc_sh,
         semi0, semi1, semg0, semg1, sems0, sems1) = refs
    sbuf = [sbuf0, sbuf1]
    dbuf = [dbuf0, dbuf1]
    rows = [rows0, rows1]
    semi = [semi0, semi1]
    semg = [semg0, semg1]
    sems = [sems0, sems1]
    if with_deg:
        semd = [semd0, semd1]
    c = lax.axis_index("c")
    s = lax.axis_index("s")

    def for_each_chunk(fn):
        def body(g, _):
            cid = s + g * _NS
            @pl.when(cid < _NCHK)
            def _():
                fn(cid * _DUMP)
            return 0
        lax.fori_loop(0, (_NCHK + _NS - 1) // _NS, body, 0)

    _zero_rows(rows0, _CH, 128)
    zslice = rows0.at[pl.ds(0, _DUMP)]
    if with_deg:
        for k in range(_DCH // 16):
            degbuf[pl.ds(k * 16, 16)] = jnp.zeros((16,), jnp.float32)
        for k in range(_CH // 16):
            onesb[pl.ds(k * 16, 16)] = jnp.ones((16,), jnp.float32)
        pltpu.sync_copy(degbuf, deg_sh.at[pl.ds(s * _DCH, _DCH)])

    def zero_chunk(off):
        pltpu.sync_copy(zslice, acc_sh.at[pl.ds(off, _DUMP)])
    for_each_chunk(zero_chunk)
    plsc.subcore_barrier()

    edges_per_core = _EPAD // 2 if split else _EPAD
    edges_per_worker = edges_per_core // _NS
    nsuper = edges_per_worker // (8 * _CH)
    cc = c if _force_half is None else _force_half
    base_edge = (cc * edges_per_core if split else 0) + s * edges_per_worker

    def run_edges(tab):
        def idx_load(m, b):
            e0 = base_edge + m * 8 * _CH
            pltpu.async_copy(src.at[pl.ds(e0, 8 * _CH)], sbuf[b], semi[b])
            pltpu.async_copy(dst.at[pl.ds(e0, 8 * _CH)], dbuf[b], semi[b])

        def idx_wait(m, b):
            e0 = base_edge + m * 8 * _CH
            pltpu.make_async_copy(src.at[pl.ds(e0, 8 * _CH)], sbuf[b], semi[b]).wait()
            pltpu.make_async_copy(dst.at[pl.ds(e0, 8 * _CH)], dbuf[b], semi[b]).wait()

        idx_load(0, 0)

        def super_body(m2, _):
            for b in range(2):
                m = m2 * 2 + b
                idx_wait(m, b)
                @pl.when(m < nsuper - 1)
                def _():
                    idx_load(m + 1, 1 - b)
                descs = {}
                for j in range(8):
                    rs = j % 2
                    if j >= 2:
                        descs[("s", rs)].wait()
                        if with_deg:
                            descs[("d", rs)].wait()
                    descs[("g", rs)] = pltpu.async_copy(
                        tab.at[sbuf[b].at[pl.ds(j * _CH, _CH)]], rows[rs],
                        semg[rs])
                    if j >= 1:
                        po = 1 - rs
                        descs[("g", po)].wait()
                        descs[("s", po)] = pltpu.async_copy(
                            rows[po],
                            acc_sh.at[dbuf[b].at[pl.ds((j - 1) * _CH, _CH)]],
                            sems[po], add=True)
                        if with_deg:
                            descs[("d", po)] = pltpu.async_copy(
                                onesb,
                                deg_sh.at[dbuf[b].at[pl.ds((j - 1) * _CH, _CH)]],
                                semd[po], add=True)
                descs[("g", 1)].wait()
                descs[("s", 1)] = pltpu.async_copy(
                    rows[1], acc_sh.at[dbuf[b].at[pl.ds(7 * _CH, _CH)]],
                    sems[1], add=True)
                if with_deg:
                    descs[("d", 1)] = pltpu.async_copy(
                        onesb, deg_sh.at[dbuf[b].at[pl.ds(7 * _CH, _CH)]],
                        semd[1], add=True)
                descs[("s", 0)].wait()
                descs[("s", 1)].wait()
                if with_deg:
                    descs[("d", 0)].wait()
                    descs[("d", 1)].wait()
            return 0
        lax.fori_loop(0, nsuper // 2, super_body, 0)

    if split:
        run_edges(tab_a)
    else:
        @pl.when(c == 0)
        def _():
            run_edges(tab_a)
        @pl.when(c == 1)
        def _():
            run_edges(tab_b)

    plsc.subcore_barrier()

    def dump_chunk(off):
        pltpu.sync_copy(acc_sh.at[pl.ds(off, _DUMP)], zslice)
        pltpu.sync_copy(zslice, acc_out.at[c].at[pl.ds(off, _DUMP)])
    for_each_chunk(dump_chunk)
    if with_deg:
        pltpu.sync_copy(deg_sh.at[pl.ds(s * _DCH, _DCH)], degbuf)
        pltpu.sync_copy(degbuf, deg_out.at[c].at[pl.ds(s * _DCH, _DCH)])


def _make_seg_sum(split, with_deg, _force_half=None):
    mesh = plsc.VectorSubcoreMesh(
        core_axis_name="c", subcore_axis_name="s",
        num_cores=_NC, num_subcores=_NS)
    out_type = [jax.ShapeDtypeStruct((_NC, _N, 128), jnp.float32)]
    scratch = [
        pltpu.VMEM((8 * _CH,), jnp.int32),
        pltpu.VMEM((8 * _CH,), jnp.int32),
        pltpu.VMEM((8 * _CH,), jnp.int32),
        pltpu.VMEM((8 * _CH,), jnp.int32),
        pltpu.VMEM((_CH, 128), jnp.float32),
        pltpu.VMEM((_CH, 128), jnp.float32),
    ]
    if with_deg:
        out_type.append(jax.ShapeDtypeStruct((_NC, _NPAD), jnp.float32))
        scratch.append(pltpu.VMEM((_CH,), jnp.float32))
        scratch.append(pltpu.VMEM((_DCH,), jnp.float32))
    scratch.append(pltpu.VMEM_SHARED((_NACC, 128), jnp.float32))
    if with_deg:
        scratch.append(pltpu.VMEM_SHARED((_NPAD,), jnp.float32))
    nsem = 8 if with_deg else 6
    scratch.extend([pltpu.SemaphoreType.DMA] * nsem)
    return pl.kernel(
        functools.partial(_seg_sum_body, split, with_deg,
                          _force_half=_force_half),
        out_type=tuple(out_type) if len(out_type) > 1 else out_type[0],
        mesh=mesh,
        scratch_types=tuple(scratch),
        name=f"sc_seg_sum_split{int(split)}_deg{int(with_deg)}",
    )


def _pair_gather_body(emb, idx, out, sidx, rows, gsem):
    c = lax.axis_index("c")
    s = lax.axis_index("s")
    wid = s * _NC + c
    rows_per_worker = (2 * _P) // (_NC * _NS)
    ch = 64
    def body(g, _):
        off = wid * rows_per_worker + g * ch
        pltpu.sync_copy(idx.at[pl.ds(off, ch)], sidx)
        pltpu.async_copy(emb.at[sidx], rows, gsem).wait()
        pltpu.sync_copy(rows, out.at[pl.ds(off, ch)])
        return 0
    lax.fori_loop(0, rows_per_worker // ch, body, 0)


def _make_pair_gather():
    mesh = plsc.VectorSubcoreMesh(
        core_axis_name="c", subcore_axis_name="s",
        num_cores=_NC, num_subcores=_NS)
    return pl.kernel(
        _pair_gather_body,
        out_type=jax.ShapeDtypeStruct((2 * _P, 128), jnp.float32),
        mesh=mesh,
        scratch_types=(
            pltpu.VMEM((64,), jnp.int32),
            pltpu.VMEM((64, 128), jnp.float32),
            pltpu.SemaphoreType.DMA,
        ),
        name="sc_pair_gather",
    )


_BR = 1000
_G = _N // _BR


def _tc_layer_in(acc_ref, degp_ref, x_ref, wl_ref, wr_ref, b_ref,
                 hpre_ref, dinv_ref, stats_ref, sacc_ref):
    i = pl.program_id(0)
    deg = degp_ref[0] + degp_ref[1]
    dinv = 1.0 / jnp.maximum(deg, 1.0)
    agg = (acc_ref[0] + acc_ref[1]) * dinv
    hpre = (jnp.dot(agg, wl_ref[...], preferred_element_type=jnp.float32)
            + jnp.dot(x_ref[...], wr_ref[...], preferred_element_type=jnp.float32)
            + b_ref[...])
    hpre_ref[...] = hpre
    dinv_ref[...] = dinv
    @pl.when(i == 0)
    def _():
        sacc_ref[...] = jnp.zeros_like(sacc_ref)
    sacc_ref[0, :] += jnp.sum(hpre, axis=0)
    sacc_ref[1, :] += jnp.sum(hpre * hpre, axis=0)
    @pl.when(i == _G - 1)
    def _():
        stats_ref[...] = sacc_ref[...]


def _tc_layer_mid(acc_ref, dinv_ref, h_ref, wl_ref, wr_ref, b_ref,
                  hpre_ref, stats_ref, sacc_ref):
    i = pl.program_id(0)
    dinv = dinv_ref[...]
    a0 = acc_ref[0] * dinv
    a1 = acc_ref[1] * dinv
    hpre = (jnp.dot(a0, wl_ref[0:128, :], preferred_element_type=jnp.float32)
            + jnp.dot(a1, wl_ref[128:256, :], preferred_element_type=jnp.float32)
            + jnp.dot(h_ref[0], wr_ref[0:128, :], preferred_element_type=jnp.float32)
            + jnp.dot(h_ref[1], wr_ref[128:256, :], preferred_element_type=jnp.float32)
            + b_ref[...])
    hpre_ref[...] = hpre
    @pl.when(i == 0)
    def _():
        sacc_ref[...] = jnp.zeros_like(sacc_ref)
    sacc_ref[0, :] += jnp.sum(hpre, axis=0)
    sacc_ref[1, :] += jnp.sum(hpre * hpre, axis=0)
    @pl.when(i == _G - 1)
    def _():
        stats_ref[...] = sacc_ref[...]


def _tc_layer_out(acc_ref, dinv_ref, h_ref, wr_ref, b_ref,
                  hpre_ref, stats_ref, sacc_ref):
    i = pl.program_id(0)
    agg = (acc_ref[0] + acc_ref[1]) * dinv_ref[...]
    hpre = (agg
            + jnp.dot(h_ref[0], wr_ref[0:128, :], preferred_element_type=jnp.float32)
            + jnp.dot(h_ref[1], wr_ref[128:256, :], preferred_element_type=jnp.float32)
            + b_ref[...])
    hpre_ref[...] = hpre
    @pl.when(i == 0)
    def _():
        sacc_ref[...] = jnp.zeros_like(sacc_ref)
    sacc_ref[0, :] += jnp.sum(hpre, axis=0)
    sacc_ref[1, :] += jnp.sum(hpre * hpre, axis=0)
    @pl.when(i == _G - 1)
    def _():
        stats_ref[...] = sacc_ref[...]


def _tc_bn_relu_split(hpre_ref, stats_ref, g_ref, be_ref, out_ref):
    mean = stats_ref[0:1, :] / _N
    var = stats_ref[1:2, :] / _N - mean * mean
    scale = g_ref[...] * lax.rsqrt(var + _EPS_BN)
    shift = be_ref[...] - mean * scale
    h = jnp.maximum(hpre_ref[...] * scale + shift, 0.0)
    out_ref[0] = h[:, 0:128]
    out_ref[1] = h[:, 128:256]


def _tc_bn_relu_split_proj(hpre_ref, stats_ref, g_ref, be_ref, wl_ref,
                           out_ref, p_ref):
    mean = stats_ref[0:1, :] / _N
    var = stats_ref[1:2, :] / _N - mean * mean
    scale = g_ref[...] * lax.rsqrt(var + _EPS_BN)
    shift = be_ref[...] - mean * scale
    h = jnp.maximum(hpre_ref[...] * scale + shift, 0.0)
    out_ref[0] = h[:, 0:128]
    out_ref[1] = h[:, 128:256]
    p_ref[...] = jnp.dot(h, wl_ref[...], preferred_element_type=jnp.float32)


def _tc_bn_l2norm(hpre_ref, stats_ref, g_ref, be_ref, emb_ref):
    mean = stats_ref[0:1, :] / _N
    var = stats_ref[1:2, :] / _N - mean * mean
    scale = g_ref[...] * lax.rsqrt(var + _EPS_BN)
    shift = be_ref[...] - mean * scale
    z = hpre_ref[...] * scale + shift
    rn = jnp.sqrt(jnp.sum(z * z, axis=-1, keepdims=True))
    emb_ref[...] = z / jnp.maximum(rn, _EPS_NORM)


def _tc_scores(a_ref, b_ref, s_ref):
    s_ref[...] = jnp.sum(a_ref[...] * b_ref[...], axis=-1, keepdims=True)


def _row_spec(shape3=None, d=256):
    return pl.BlockSpec((_BR, d), lambda i: (i, 0))


def kernel(x, edge_index, src_indices, dst_indices,
           Wl0, Wr0, b0, g0, be0,
           Wl1, Wr1, b1, g1, be1,
           Wl2, Wr2, b2, g2, be2):
    src = edge_index[0].astype(jnp.int32)
    dst = edge_index[1].astype(jnp.int32)
    npad = _EPAD - _E
    pad_src = jnp.arange(npad, dtype=jnp.int32) % _N
    src1d = jnp.concatenate([src, pad_src])
    pad_dst = _N + (jnp.arange(npad, dtype=jnp.int32) % (_NACC - _N))
    dst1d = jnp.concatenate([dst, pad_dst])
    pair_idx = jnp.concatenate([src_indices.astype(jnp.int32),
                                dst_indices.astype(jnp.int32)])
    b0r, g0r, be0r = b0.reshape(1, -1), g0.reshape(1, -1), be0.reshape(1, -1)
    b1r, g1r, be1r = b1.reshape(1, -1), g1.reshape(1, -1), be1.reshape(1, -1)
    b2r, g2r, be2r = b2.reshape(1, -1), g2.reshape(1, -1), be2.reshape(1, -1)

    seg_deg = _make_seg_sum(split=True, with_deg=True)
    seg_split = _make_seg_sum(split=True, with_deg=False)
    seg_stack = _make_seg_sum(split=False, with_deg=False)
    pair_gather = _make_pair_gather()

    full = lambda shp: pl.BlockSpec(shp, lambda i: tuple(0 for _ in shp))
    acc_spec = pl.BlockSpec((2, _BR, 128), lambda i: (0, i, 0))
    h2_spec = pl.BlockSpec((2, _BR, 128), lambda i: (0, i, 0))
    cp_arb = pltpu.CompilerParams(dimension_semantics=("arbitrary",))

    acc0, degp = seg_deg(x, x, src1d, dst1d)
    degp = degp[:, :_N].reshape(2, _N, 1)

    hpre1, dinv, stats1 = pl.pallas_call(
        _tc_layer_in,
        grid=(_G,),
        in_specs=[acc_spec,
                  pl.BlockSpec((2, _BR, 1), lambda i: (0, i, 0)),
                  pl.BlockSpec((_BR, 128), lambda i: (i, 0)),
                  full((128, 256)), full((128, 256)), full((1, 256))],
        out_specs=[pl.BlockSpec((_BR, 256), lambda i: (i, 0)),
                   pl.BlockSpec((_BR, 1), lambda i: (i, 0)),
                   full((2, 256))],
        out_shape=[jax.ShapeDtypeStruct((_N, 256), jnp.float32),
                   jax.ShapeDtypeStruct((_N, 1), jnp.float32),
                   jax.ShapeDtypeStruct((2, 256), jnp.float32)],
        scratch_shapes=[pltpu.VMEM((2, 256), jnp.float32)],
        compiler_params=cp_arb,
        name="tc_layer0",
    )(acc0, degp, x, Wl0, Wr0, b0r)

    h1s = pl.pallas_call(
        _tc_bn_relu_split,
        grid=(_G,),
        in_specs=[pl.BlockSpec((_BR, 256), lambda i: (i, 0)),
                  full((2, 256)), full((1, 256)), full((1, 256))],
        out_specs=h2_spec,
        out_shape=jax.ShapeDtypeStruct((2, _N, 128), jnp.float32),
        name="tc_bn0",
    )(hpre1, stats1, g0r, be0r)

    acc1 = seg_stack(h1s[0], h1s[1], src1d, dst1d)

    hpre2, stats2 = pl.pallas_call(
        _tc_layer_mid,
        grid=(_G,),
        in_specs=[acc_spec,
                  pl.BlockSpec((_BR, 1), lambda i: (i, 0)),
                  h2_spec,
                  full((256, 256)), full((256, 256)), full((1, 256))],
        out_specs=[pl.BlockSpec((_BR, 256), lambda i: (i, 0)),
                   full((2, 256))],
        out_shape=[jax.ShapeDtypeStruct((_N, 256), jnp.float32),
                   jax.ShapeDtypeStruct((2, 256), jnp.float32)],
        scratch_shapes=[pltpu.VMEM((2, 256), jnp.float32)],
        compiler_params=cp_arb,
        name="tc_layer1",
    )(acc1, dinv, h1s, Wl1, Wr1, b1r)

    h2s, p = pl.pallas_call(
        _tc_bn_relu_split_proj,
        grid=(_G,),
        in_specs=[pl.BlockSpec((_BR, 256), lambda i: (i, 0)),
                  full((2, 256)), full((1, 256)), full((1, 256)),
                  full((256, 128))],
        out_specs=[h2_spec, pl.BlockSpec((_BR, 128), lambda i: (i, 0))],
        out_shape=[jax.ShapeDtypeStruct((2, _N, 128), jnp.float32),
                   jax.ShapeDtypeStruct((_N, 128), jnp.float32)],
        name="tc_bn1",
    )(hpre2, stats2, g1r, be1r, Wl2)

    acc2 = seg_split(p, p, src1d, dst1d)

    hpre3, stats3 = pl.pallas_call(
        _tc_layer_out,
        grid=(_G,),
        in_specs=[acc_spec,
                  pl.BlockSpec((_BR, 1), lambda i: (i, 0)),
                  h2_spec,
                  full((256, 128)), full((1, 128))],
        out_specs=[pl.BlockSpec((_BR, 128), lambda i: (i, 0)),
                   full((2, 128))],
        out_shape=[jax.ShapeDtypeStruct((_N, 128), jnp.float32),
                   jax.ShapeDtypeStruct((2, 128), jnp.float32)],
        scratch_shapes=[pltpu.VMEM((2, 128), jnp.float32)],
        compiler_params=cp_arb,
        name="tc_layer2",
    )(acc2, dinv, h2s, Wr2, b2r)

    emb = pl.pallas_call(
        _tc_bn_l2norm,
        grid=(_G,),
        in_specs=[pl.BlockSpec((_BR, 128), lambda i: (i, 0)),
                  full((2, 128)), full((1, 128)), full((1, 128))],
        out_specs=pl.BlockSpec((_BR, 128), lambda i: (i, 0)),
        out_shape=jax.ShapeDtypeStruct((_N, 128), jnp.float32),
        name="tc_bn2_l2",
    )(hpre3, stats3, g2r, be2r)

    pairs = pair_gather(emb, pair_idx)

    pb = 1024
    scores2d = pl.pallas_call(
        _tc_scores,
        grid=(_P // pb,),
        in_specs=[pl.BlockSpec((pb, 128), lambda i: (i, 0)),
                  pl.BlockSpec((pb, 128), lambda i: (i + _P // pb, 0))],
        out_specs=pl.BlockSpec((pb, 1), lambda i: (i, 0)),
        out_shape=jax.ShapeDtypeStruct((_P, 1), jnp.float32),
        name="tc_scores",
    )(pairs, pairs)
    scores = scores2d[:, 0]

    return (emb, scores)


# speedup vs baseline: 2.9911x; 1.0007x over previous
import functools

import jax
import jax.numpy as jnp
from jax import lax
from jax.experimental import pallas as pl
from jax.experimental.pallas import tpu as pltpu
from jax.experimental.pallas import tpu_sc as plsc

_N = 10000
_E = 320000
_P = 8192
_EPS_BN = 1e-5
_EPS_NORM = 1e-12

_NC = 2
_NS = 16
_CH = 128
_EPAD = 327680
_EROWS = _EPAD // _CH
_NACC = 10240
_DUMP = 80
_NCHK = _N // _DUMP
_NPAD = 10240
_DCH = _NPAD // _NS


def _zero_rows(ref, nrows, ncols):
    def body(r, _):
        for k in range(ncols // 16):
            ref[r, pl.ds(k * 16, 16)] = jnp.zeros((16,), jnp.float32)
        return 0
    lax.fori_loop(0, nrows, body, 0)


def _seg_sum_body(split, with_deg, *refs, _force_half=None):
    if with_deg:
        (tab_a, tab_b, src, dst, acc_out, deg_out,
         sbuf0, sbuf1, dbuf0, dbuf1, rows0, rows1, onesb, degbuf,
         acc_sh, deg_sh,
         semi0, semi1, semg0, semg1, sems0, sems1, semd0, semd1) = refs
    else:
        (tab_a, tab_b, src, dst, acc_out,
         sbuf0, sbuf1, dbuf0, dbuf1, rows0, rows1,
         acc_sh,
         semi0, semi1, semg0, semg1, sems0, sems1) = refs
    sbuf = [sbuf0, sbuf1]
    dbuf = [dbuf0, dbuf1]
    rows = [rows0, rows1]
    semi = [semi0, semi1]
    semg = [semg0, semg1]
    sems = [sems0, sems1]
    if with_deg:
        semd = [semd0, semd1]
    c = lax.axis_index("c")
    s = lax.axis_index("s")

    def for_each_chunk(fn):
        def body(g, _):
            cid = s + g * _NS
            @pl.when(cid < _NCHK)
            def _():
                fn(cid * _DUMP)
            return 0
        lax.fori_loop(0, (_NCHK + _NS - 1) // _NS, body, 0)

    _zero_rows(rows0, _CH, 128)
    zslice = rows0.at[pl.ds(0, _DUMP)]
    if with_deg:
        for k in range(_DCH // 16):
            degbuf[pl.ds(k * 16, 16)] = jnp.zeros((16,), jnp.float32)
        for k in range(_CH // 16):
            onesb[pl.ds(k * 16, 16)] = jnp.ones((16,), jnp.float32)
        pltpu.sync_copy(degbuf, deg_sh.at[pl.ds(s * _DCH, _DCH)])

    def zero_chunk(off):
        pltpu.sync_copy(zslice, acc_sh.at[pl.ds(off, _DUMP)])
    for_each_chunk(zero_chunk)
    plsc.subcore_barrier()

    edges_per_core = _EPAD // 2 if split else _EPAD
    edges_per_worker = edges_per_core // _NS
    nsuper = edges_per_worker // (8 * _CH)
    cc = c if _force_half is None else _force_half
    base_edge = (cc * edges_per_core if split else 0) + s * edges_per_worker

    def run_edges(tab):
        def idx_load(m, b):
            e0 = base_edge + m * 8 * _CH
            pltpu.async_copy(src.at[pl.ds(e0, 8 * _CH)], sbuf[b], semi[b])
            pltpu.async_copy(dst.at[pl.ds(e0, 8 * _CH)], dbuf[b], semi[b])

        def idx_wait(m, b):
            e0 = base_edge + m * 8 * _CH
            pltpu.make_async_copy(src.at[pl.ds(e0, 8 * _CH)], sbuf[b], semi[b]).wait()
            pltpu.make_async_copy(dst.at[pl.ds(e0, 8 * _CH)], dbuf[b], semi[b]).wait()

        idx_load(0, 0)

        def super_body(m2, _):
            for b in range(2):
                m = m2 * 2 + b
                idx_wait(m, b)
                @pl.when(m < nsuper - 1)
                def _():
                    idx_load(m + 1, 1 - b)
                descs = {}
                for j in range(8):
                    rs = j % 2
                    if j >= 2:
                        descs[("s", rs)].wait()
                        if with_deg:
                            descs[("d", rs)].wait()
                    descs[("g", rs)] = pltpu.async_copy(
                        tab.at[sbuf[b].at[pl.ds(j * _CH, _CH)]], rows[rs],
                        semg[rs])
                    if j >= 1:
                        po = 1 - rs
                        descs[("g", po)].wait()
                        descs[("s", po)] = pltpu.async_copy(
                            rows[po],
                            acc_sh.at[dbuf[b].at[pl.ds((j - 1) * _CH, _CH)]],
                            sems[po], add=True)
                        if with_deg:
                            descs[("d", po)] = pltpu.async_copy(
                                onesb,
                                deg_sh.at[dbuf[b].at[pl.ds((j - 1) * _CH, _CH)]],
                                semd[po], add=True)
                descs[("g", 1)].wait()
                descs[("s", 1)] = pltpu.async_copy(
                    rows[1], acc_sh.at[dbuf[b].at[pl.ds(7 * _CH, _CH)]],
                    sems[1], add=True)
                if with_deg:
                    descs[("d", 1)] = pltpu.async_copy(
                        onesb, deg_sh.at[dbuf[b].at[pl.ds(7 * _CH, _CH)]],
                        semd[1], add=True)
                descs[("s", 0)].wait()
                descs[("s", 1)].wait()
                if with_deg:
                    descs[("d", 0)].wait()
                    descs[("d", 1)].wait()
            return 0
        lax.fori_loop(0, nsuper // 2, super_body, 0)

    if split:
        run_edges(tab_a)
    else:
        @pl.when(c == 0)
        def _():
            run_edges(tab_a)
        @pl.when(c == 1)
        def _():
            run_edges(tab_b)

    plsc.subcore_barrier()

    def dump_chunk(off):
        pltpu.sync_copy(acc_sh.at[pl.ds(off, _DUMP)], zslice)
        pltpu.sync_copy(zslice, acc_out.at[c].at[pl.ds(off, _DUMP)])
    for_each_chunk(dump_chunk)
    if with_deg:
        pltpu.sync_copy(deg_sh.at[pl.ds(s * _DCH, _DCH)], degbuf)
        pltpu.sync_copy(degbuf, deg_out.at[c].at[pl.ds(s * _DCH, _DCH)])


def _make_seg_sum(split, with_deg, _force_half=None):
    mesh = plsc.VectorSubcoreMesh(
        core_axis_name="c", subcore_axis_name="s",
        num_cores=_NC, num_subcores=_NS)
    out_type = [jax.ShapeDtypeStruct((_NC, _N, 128), jnp.float32)]
    scratch = [
        pltpu.VMEM((8 * _CH,), jnp.int32),
        pltpu.VMEM((8 * _CH,), jnp.int32),
        pltpu.VMEM((8 * _CH,), jnp.int32),
        pltpu.VMEM((8 * _CH,), jnp.int32),
        pltpu.VMEM((_CH, 128), jnp.float32),
        pltpu.VMEM((_CH, 128), jnp.float32),
    ]
    if with_deg:
        out_type.append(jax.ShapeDtypeStruct((_NC, _NPAD), jnp.float32))
        scratch.append(pltpu.VMEM((_CH,), jnp.float32))
        scratch.append(pltpu.VMEM((_DCH,), jnp.float32))
    scratch.append(pltpu.VMEM_SHARED((_NACC, 128), jnp.float32))
    if with_deg:
        scratch.append(pltpu.VMEM_SHARED((_NPAD,), jnp.float32))
    nsem = 8 if with_deg else 6
    scratch.extend([pltpu.SemaphoreType.DMA] * nsem)
    return pl.kernel(
        functools.partial(_seg_sum_body, split, with_deg,
                          _force_half=_force_half),
        out_type=tuple(out_type) if len(out_type) > 1 else out_type[0],
        mesh=mesh,
        scratch_types=tuple(scratch),
        name=f"sc_seg_sum_split{int(split)}_deg{int(with_deg)}",
    )


def _pair_gather_body(emb, idx, out, sidx, rows, gsem):
    c = lax.axis_index("c")
    s = lax.axis_index("s")
    wid = s * _NC + c
    rows_per_worker = (2 * _P) // (_NC * _NS)
    ch = 64
    def body(g, _):
        off = wid * rows_per_worker + g * ch
        pltpu.sync_copy(idx.at[pl.ds(off, ch)], sidx)
        pltpu.async_copy(emb.at[sidx], rows, gsem).wait()
        pltpu.sync_copy(rows, out.at[pl.ds(off, ch)])
        return 0
    lax.fori_loop(0, rows_per_worker // ch, body, 0)


def _make_pair_gather():
    mesh = plsc.VectorSubcoreMesh(
        core_axis_name="c", subcore_axis_name="s",
        num_cores=_NC, num_subcores=_NS)
    return pl.kernel(
        _pair_gather_body,
        out_type=jax.ShapeDtypeStruct((2 * _P, 128), jnp.float32),
        mesh=mesh,
        scratch_types=(
            pltpu.VMEM((64,), jnp.int32),
            pltpu.VMEM((64, 128), jnp.float32),
            pltpu.SemaphoreType.DMA,
        ),
        name="sc_pair_gather",
    )


_BR = 1000
_G = _N // _BR


def _tc_xw_in(x_ref, wr_ref, b_ref, xw_ref):
    xw_ref[...] = (jnp.dot(x_ref[...], wr_ref[...],
                           preferred_element_type=jnp.float32) + b_ref[...])


def _tc_xw_mid(h_ref, wr_ref, b_ref, xw_ref):
    xw_ref[...] = (
        jnp.dot(h_ref[0], wr_ref[0:128, :], preferred_element_type=jnp.float32)
        + jnp.dot(h_ref[1], wr_ref[128:256, :],
                  preferred_element_type=jnp.float32)
        + b_ref[...])


def _stats_accum(i, hpre, stats_ref, sacc_ref):
    @pl.when(i == 0)
    def _():
        sacc_ref[...] = jnp.zeros_like(sacc_ref)
    sacc_ref[0, :] += jnp.sum(hpre, axis=0)
    sacc_ref[1, :] += jnp.sum(hpre * hpre, axis=0)
    @pl.when(i == _G - 1)
    def _():
        stats_ref[...] = sacc_ref[...]


def _tc_agg_in(acc_ref, degp_ref, xw_ref, wl_ref,
               hpre_ref, dinv_ref, stats_ref, sacc_ref):
    i = pl.program_id(0)
    deg = degp_ref[0] + degp_ref[1]
    dinv = 1.0 / jnp.maximum(deg, 1.0)
    agg = (acc_ref[0] + acc_ref[1]) * dinv
    hpre = (jnp.dot(agg, wl_ref[...], preferred_element_type=jnp.float32)
            + xw_ref[...])
    hpre_ref[...] = hpre
    dinv_ref[...] = dinv
    _stats_accum(i, hpre, stats_ref, sacc_ref)


def _tc_agg_mid(acc_ref, dinv_ref, xw_ref, wl_ref,
                hpre_ref, stats_ref, sacc_ref):
    i = pl.program_id(0)
    dinv = dinv_ref[...]
    a0 = acc_ref[0] * dinv
    a1 = acc_ref[1] * dinv
    hpre = (jnp.dot(a0, wl_ref[0:128, :], preferred_element_type=jnp.float32)
            + jnp.dot(a1, wl_ref[128:256, :],
                      preferred_element_type=jnp.float32)
            + xw_ref[...])
    hpre_ref[...] = hpre
    _stats_accum(i, hpre, stats_ref, sacc_ref)


def _tc_agg_out(acc_ref, dinv_ref, xw_ref, hpre_ref, stats_ref, sacc_ref):
    i = pl.program_id(0)
    hpre = (acc_ref[0] + acc_ref[1]) * dinv_ref[...] + xw_ref[...]
    hpre_ref[...] = hpre
    _stats_accum(i, hpre, stats_ref, sacc_ref)


def _tc_bn_relu_split(hpre_ref, stats_ref, g_ref, be_ref, out_ref):
    mean = stats_ref[0:1, :] / _N
    var = stats_ref[1:2, :] / _N - mean * mean
    scale = g_ref[...] * lax.rsqrt(var + _EPS_BN)
    shift = be_ref[...] - mean * scale
    h = jnp.maximum(hpre_ref[...] * scale + shift, 0.0)
    out_ref[0] = h[:, 0:128]
    out_ref[1] = h[:, 128:256]


def _tc_bn_relu_split_proj(hpre_ref, stats_ref, g_ref, be_ref, wl_ref,
                           out_ref, p_ref):
    mean = stats_ref[0:1, :] / _N
    var = stats_ref[1:2, :] / _N - mean * mean
    scale = g_ref[...] * lax.rsqrt(var + _EPS_BN)
    shift = be_ref[...] - mean * scale
    h = jnp.maximum(hpre_ref[...] * scale + shift, 0.0)
    out_ref[0] = h[:, 0:128]
    out_ref[1] = h[:, 128:256]
    p_ref[...] = jnp.dot(h, wl_ref[...], preferred_element_type=jnp.float32)


def _tc_bn_l2norm(hpre_ref, stats_ref, g_ref, be_ref, emb_ref):
    mean = stats_ref[0:1, :] / _N
    var = stats_ref[1:2, :] / _N - mean * mean
    scale = g_ref[...] * lax.rsqrt(var + _EPS_BN)
    shift = be_ref[...] - mean * scale
    z = hpre_ref[...] * scale + shift
    rn = jnp.sqrt(jnp.sum(z * z, axis=-1, keepdims=True))
    emb_ref[...] = z / jnp.maximum(rn, _EPS_NORM)


def _tc_scores(a_ref, b_ref, s_ref):
    s_ref[...] = jnp.sum(a_ref[...] * b_ref[...], axis=-1, keepdims=True)


def _row_spec(shape3=None, d=256):
    return pl.BlockSpec((_BR, d), lambda i: (i, 0))


def kernel(x, edge_index, src_indices, dst_indices,
           Wl0, Wr0, b0, g0, be0,
           Wl1, Wr1, b1, g1, be1,
           Wl2, Wr2, b2, g2, be2):
    src = edge_index[0].astype(jnp.int32)
    dst = edge_index[1].astype(jnp.int32)
    npad = _EPAD - _E
    pad_src = jnp.arange(npad, dtype=jnp.int32) % _N
    src1d = jnp.concatenate([src, pad_src])
    pad_dst = _N + (jnp.arange(npad, dtype=jnp.int32) % (_NACC - _N))
    dst1d = jnp.concatenate([dst, pad_dst])
    pair_idx = jnp.concatenate([src_indices.astype(jnp.int32),
                                dst_indices.astype(jnp.int32)])
    b0r, g0r, be0r = b0.reshape(1, -1), g0.reshape(1, -1), be0.reshape(1, -1)
    b1r, g1r, be1r = b1.reshape(1, -1), g1.reshape(1, -1), be1.reshape(1, -1)
    b2r, g2r, be2r = b2.reshape(1, -1), g2.reshape(1, -1), be2.reshape(1, -1)

    seg_deg = _make_seg_sum(split=True, with_deg=True)
    seg_split = _make_seg_sum(split=True, with_deg=False)
    seg_stack = _make_seg_sum(split=False, with_deg=False)
    pair_gather = _make_pair_gather()

    full = lambda shp: pl.BlockSpec(shp, lambda i: tuple(0 for _ in shp))
    acc_spec = pl.BlockSpec((2, _BR, 128), lambda i: (0, i, 0))
    h2_spec = pl.BlockSpec((2, _BR, 128), lambda i: (0, i, 0))
    cp_arb = pltpu.CompilerParams(dimension_semantics=("arbitrary",))

    acc0, degp = seg_deg(x, x, src1d, dst1d)
    degp = degp[:, :_N].reshape(2, _N, 1)

    xw0 = pl.pallas_call(
        _tc_xw_in,
        grid=(_G,),
        in_specs=[pl.BlockSpec((_BR, 128), lambda i: (i, 0)),
                  full((128, 256)), full((1, 256))],
        out_specs=pl.BlockSpec((_BR, 256), lambda i: (i, 0)),
        out_shape=jax.ShapeDtypeStruct((_N, 256), jnp.float32),
        name="tc_xw0",
    )(x, Wr0, b0r)

    hpre1, dinv, stats1 = pl.pallas_call(
        _tc_agg_in,
        grid=(_G,),
        in_specs=[acc_spec,
                  pl.BlockSpec((2, _BR, 1), lambda i: (0, i, 0)),
                  pl.BlockSpec((_BR, 256), lambda i: (i, 0)),
                  full((128, 256))],
        out_specs=[pl.BlockSpec((_BR, 256), lambda i: (i, 0)),
                   pl.BlockSpec((_BR, 1), lambda i: (i, 0)),
                   full((2, 256))],
        out_shape=[jax.ShapeDtypeStruct((_N, 256), jnp.float32),
                   jax.ShapeDtypeStruct((_N, 1), jnp.float32),
                   jax.ShapeDtypeStruct((2, 256), jnp.float32)],
        scratch_shapes=[pltpu.VMEM((2, 256), jnp.float32)],
        compiler_params=cp_arb,
        name="tc_agg0",
    )(acc0, degp, xw0, Wl0)

    h1s = pl.pallas_call(
        _tc_bn_relu_split,
        grid=(_G,),
        in_specs=[pl.BlockSpec((_BR, 256), lambda i: (i, 0)),
                  full((2, 256)), full((1, 256)), full((1, 256))],
        out_specs=h2_spec,
        out_shape=jax.ShapeDtypeStruct((2, _N, 128), jnp.float32),
        name="tc_bn0",
    )(hpre1, stats1, g0r, be0r)

    acc1 = seg_stack(h1s[0], h1s[1], src1d, dst1d)

    xw1 = pl.pallas_call(
        _tc_xw_mid,
        grid=(_G,),
        in_specs=[h2_spec, full((256, 256)), full((1, 256))],
        out_specs=pl.BlockSpec((_BR, 256), lambda i: (i, 0)),
        out_shape=jax.ShapeDtypeStruct((_N, 256), jnp.float32),
        name="tc_xw1",
    )(h1s, Wr1, b1r)

    hpre2, stats2 = pl.pallas_call(
        _tc_agg_mid,
        grid=(_G,),
        in_specs=[acc_spec,
                  pl.BlockSpec((_BR, 1), lambda i: (i, 0)),
                  pl.BlockSpec((_BR, 256), lambda i: (i, 0)),
                  full((256, 256))],
        out_specs=[pl.BlockSpec((_BR, 256), lambda i: (i, 0)),
                   full((2, 256))],
        out_shape=[jax.ShapeDtypeStruct((_N, 256), jnp.float32),
                   jax.ShapeDtypeStruct((2, 256), jnp.float32)],
        scratch_shapes=[pltpu.VMEM((2, 256), jnp.float32)],
        compiler_params=cp_arb,
        name="tc_agg1",
    )(acc1, dinv, xw1, Wl1)

    h2s, p = pl.pallas_call(
        _tc_bn_relu_split_proj,
        grid=(_G,),
        in_specs=[pl.BlockSpec((_BR, 256), lambda i: (i, 0)),
                  full((2, 256)), full((1, 256)), full((1, 256)),
                  full((256, 128))],
        out_specs=[h2_spec, pl.BlockSpec((_BR, 128), lambda i: (i, 0))],
        out_shape=[jax.ShapeDtypeStruct((2, _N, 128), jnp.float32),
                   jax.ShapeDtypeStruct((_N, 128), jnp.float32)],
        name="tc_bn1",
    )(hpre2, stats2, g1r, be1r, Wl2)

    acc2 = seg_split(p, p, src1d, dst1d)

    xw2 = pl.pallas_call(
        _tc_xw_mid,
        grid=(_G,),
        in_specs=[h2_spec, full((256, 128)), full((1, 128))],
        out_specs=pl.BlockSpec((_BR, 128), lambda i: (i, 0)),
        out_shape=jax.ShapeDtypeStruct((_N, 128), jnp.float32),
        name="tc_xw2",
    )(h2s, Wr2, b2r)

    hpre3, stats3 = pl.pallas_call(
        _tc_agg_out,
        grid=(_G,),
        in_specs=[acc_spec,
                  pl.BlockSpec((_BR, 1), lambda i: (i, 0)),
                  pl.BlockSpec((_BR, 128), lambda i: (i, 0))],
        out_specs=[pl.BlockSpec((_BR, 128), lambda i: (i, 0)),
                   full((2, 128))],
        out_shape=[jax.ShapeDtypeStruct((_N, 128), jnp.float32),
                   jax.ShapeDtypeStruct((2, 128), jnp.float32)],
        scratch_shapes=[pltpu.VMEM((2, 128), jnp.float32)],
        compiler_params=cp_arb,
        name="tc_agg2",
    )(acc2, dinv, xw2)

    emb = pl.pallas_call(
        _tc_bn_l2norm,
        grid=(_G,),
        in_specs=[pl.BlockSpec((_BR, 128), lambda i: (i, 0)),
                  full((2, 128)), full((1, 128)), full((1, 128))],
        out_specs=pl.BlockSpec((_BR, 128), lambda i: (i, 0)),
        out_shape=jax.ShapeDtypeStruct((_N, 128), jnp.float32),
        name="tc_bn2_l2",
    )(hpre3, stats3, g2r, be2r)

    pairs = pair_gather(emb, pair_idx)

    pb = 1024
    scores2d = pl.pallas_call(
        _tc_scores,
        grid=(_P // pb,),
        in_specs=[pl.BlockSpec((pb, 128), lambda i: (i, 0)),
                  pl.BlockSpec((pb, 128), lambda i: (i + _P // pb, 0))],
        out_specs=pl.BlockSpec((pb, 1), lambda i: (i, 0)),
        out_shape=jax.ShapeDtypeStruct((_P, 1), jnp.float32),
        name="tc_scores",
    )(pairs, pairs)
    scores = scores2d[:, 0]

    return (emb, scores)
